# baseline (speedup 1.0000x reference)
import functools

import jax
import jax.numpy as jnp
from jax import lax
from jax.experimental import pallas as pl
from jax.experimental.pallas import tpu as pltpu, tpu_sc as plsc


_NC = 2
_NS = 16
_NW = _NC * _NS
_CHUNK = 8
_LANES = 16


def kernel(tensor_in, pos_table):
    batch, seq, dim = tensor_in.shape
    rows = batch * seq
    per_w = rows // _NW
    x2d = tensor_in.reshape(rows, dim)
    mesh = plsc.VectorSubcoreMesh(core_axis_name="c", subcore_axis_name="s")

    @functools.partial(
        pl.kernel,
        out_type=jax.ShapeDtypeStruct((rows, dim), jnp.float32),
        mesh=mesh,
        scratch_types=[
            pltpu.VMEM((_CHUNK, dim), jnp.float32),
            pltpu.VMEM((_CHUNK, dim), jnp.float32),
        ],
    )
    def sc_add(x_hbm, tab_hbm, out_hbm, tbuf, pbuf):
        wid = lax.axis_index("s") * _NC + lax.axis_index("c")
        base = wid * per_w
        sbase = base % seq

        def chunk_body(k, carry):
            r = base + k * _CHUNK
            sr = sbase + k * _CHUNK
            pltpu.sync_copy(x_hbm.at[pl.ds(r, _CHUNK)], tbuf)
            pltpu.sync_copy(tab_hbm.at[pl.ds(sr, _CHUNK)], pbuf)
            for row in range(_CHUNK):
                def vec_body(i, c2):
                    j = i * _LANES
                    plsc.addupdate(
                        tbuf.at[row, pl.ds(j, _LANES)],
                        pbuf[row, pl.ds(j, _LANES)],
                    )
                    return c2
                lax.fori_loop(0, dim // _LANES, vec_body, 0)
            pltpu.sync_copy(tbuf, out_hbm.at[pl.ds(r, _CHUNK)])
            return carry

        lax.fori_loop(0, per_w // _CHUNK, chunk_body, 0)

    out2d = sc_add(x2d, pos_table)
    return out2d.reshape(batch, seq, dim)


# speedup vs baseline: 1.3741x; 1.3741x over previous
import functools

import jax
import jax.numpy as jnp
from jax import lax
from jax.experimental import pallas as pl
from jax.experimental.pallas import tpu as pltpu, tpu_sc as plsc


_NC = 2
_NS = 16
_NW = _NC * _NS
_CHUNK = 8
_LANES = 16


def kernel(tensor_in, pos_table):
    batch, seq, dim = tensor_in.shape
    rows = batch * seq
    per_w = rows // _NW
    nchunks = per_w // _CHUNK
    x2d = tensor_in.reshape(rows, dim)
    mesh = plsc.VectorSubcoreMesh(core_axis_name="c", subcore_axis_name="s")

    @functools.partial(
        pl.kernel,
        out_type=jax.ShapeDtypeStruct((rows, dim), jnp.float32),
        mesh=mesh,
        scratch_types=[
            pltpu.VMEM((2, _CHUNK, dim), jnp.float32),
            pltpu.VMEM((2, _CHUNK, dim), jnp.float32),
            pltpu.SemaphoreType.DMA,
            pltpu.SemaphoreType.DMA,
            pltpu.SemaphoreType.DMA,
            pltpu.SemaphoreType.DMA,
            pltpu.SemaphoreType.DMA,
            pltpu.SemaphoreType.DMA,
        ],
    )
    def sc_add(x_hbm, tab_hbm, out_hbm, tbuf, pbuf,
               ts0, ts1, ps0, ps1, os0, os1):
        tsem = (ts0, ts1)
        psem = (ps0, ps1)
        osem = (os0, os1)
        wid = lax.axis_index("s") * _NC + lax.axis_index("c")
        base = wid * per_w
        sbase = base % seq

        def start_in(k):
            s = k % 2
            r = base + k * _CHUNK
            sr = sbase + k * _CHUNK
            dt = pltpu.async_copy(x_hbm.at[pl.ds(r, _CHUNK)], tbuf.at[s], tsem[s])
            dp = pltpu.async_copy(tab_hbm.at[pl.ds(sr, _CHUNK)], pbuf.at[s], psem[s])
            return dt, dp

        ind = [None] * nchunks
        outd = [None] * nchunks
        ind[0] = start_in(0)
        for k in range(nchunks):
            s = k % 2
            if k + 1 < nchunks:
                if k - 1 >= 0:
                    outd[k - 1].wait()
                ind[k + 1] = start_in(k + 1)
            ind[k][0].wait()
            ind[k][1].wait()
            for row in range(_CHUNK):
                def vec_body(i, c2, _s=s, _row=row):
                    j = i * _LANES
                    plsc.addupdate(
                        tbuf.at[_s, _row, pl.ds(j, _LANES)],
                        pbuf[_s, _row, pl.ds(j, _LANES)],
                    )
                    return c2
                lax.fori_loop(0, dim // _LANES, vec_body, 0)
            r = base + k * _CHUNK
            outd[k] = pltpu.async_copy(tbuf.at[s], out_hbm.at[pl.ds(r, _CHUNK)], osem[s])
        outd[nchunks - 2].wait()
        outd[nchunks - 1].wait()

    out2d = sc_add(x2d, pos_table)
    return out2d.reshape(batch, seq, dim)


# speedup vs baseline: 2.0833x; 1.5161x over previous
import functools

import jax
import jax.numpy as jnp
from jax import lax
from jax.experimental import pallas as pl
from jax.experimental.pallas import tpu as pltpu, tpu_sc as plsc


_NC = 2
_NS = 16
_NW = _NC * _NS
_CHUNK = 8
_LANES = 16


def kernel(tensor_in, pos_table):
    batch, seq, dim = tensor_in.shape
    rows = batch * seq
    per_w = rows // _NW
    nchunks = per_w // _CHUNK
    x2d = tensor_in.reshape(rows, dim)
    mesh = plsc.VectorSubcoreMesh(core_axis_name="c", subcore_axis_name="s")

    @functools.partial(
        pl.kernel,
        out_type=jax.ShapeDtypeStruct((rows, dim), jnp.float32),
        mesh=mesh,
        scratch_types=[
            pltpu.VMEM((2, _CHUNK, dim), jnp.float32),
            pltpu.VMEM((2, _CHUNK, dim), jnp.float32),
            pltpu.SemaphoreType.DMA,
            pltpu.SemaphoreType.DMA,
            pltpu.SemaphoreType.DMA,
            pltpu.SemaphoreType.DMA,
            pltpu.SemaphoreType.DMA,
            pltpu.SemaphoreType.DMA,
        ],
    )
    def sc_add(x_hbm, tab_hbm, out_hbm, tbuf, pbuf,
               ts0, ts1, ps0, ps1, os0, os1):
        tsem = (ts0, ts1)
        psem = (ps0, ps1)
        osem = (os0, os1)
        wid = lax.axis_index("s") * _NC + lax.axis_index("c")
        base = wid * per_w
        sbase = base % seq

        def start_in(k):
            s = k % 2
            r = base + k * _CHUNK
            sr = sbase + k * _CHUNK
            dt = pltpu.async_copy(x_hbm.at[pl.ds(r, _CHUNK)], tbuf.at[s], tsem[s])
            dp = pltpu.async_copy(tab_hbm.at[pl.ds(sr, _CHUNK)], pbuf.at[s], psem[s])
            return dt, dp

        ind = [None] * nchunks
        outd = [None] * nchunks
        ind[0] = start_in(0)
        for k in range(nchunks):
            s = k % 2
            if k + 1 < nchunks:
                if k - 1 >= 0:
                    outd[k - 1].wait()
                ind[k + 1] = start_in(k + 1)
            ind[k][0].wait()
            ind[k][1].wait()
            for row in range(_CHUNK):
                @plsc.parallel_loop(0, dim, _LANES, unroll=8)
                def _vec_body(j, _s=s, _row=row):
                    plsc.addupdate(
                        tbuf.at[_s, _row, pl.ds(j, _LANES)],
                        pbuf[_s, _row, pl.ds(j, _LANES)],
                    )
            r = base + k * _CHUNK
            outd[k] = pltpu.async_copy(tbuf.at[s], out_hbm.at[pl.ds(r, _CHUNK)], osem[s])
        outd[nchunks - 2].wait()
        outd[nchunks - 1].wait()

    out2d = sc_add(x2d, pos_table)
    return out2d.reshape(batch, seq, dim)


# speedup vs baseline: 5.1919x; 2.4921x over previous
import jax
import jax.numpy as jnp
from jax.experimental import pallas as pl
from jax.experimental.pallas import tpu as pltpu


_SEQ_BLOCK = 1024


def _add_block(tensor_ref, pos_ref, out_ref):
    out_ref[...] = tensor_ref[...] + pos_ref[...]


def kernel(tensor_in, pos_table):
    batch, seq, dim = tensor_in.shape
    grid = (seq // _SEQ_BLOCK, batch)
    return pl.pallas_call(
        _add_block,
        grid=grid,
        in_specs=[
            pl.BlockSpec((1, _SEQ_BLOCK, dim), lambda i, j: (j, i, 0)),
            pl.BlockSpec((_SEQ_BLOCK, dim), lambda i, j: (i, 0)),
        ],
        out_specs=pl.BlockSpec((1, _SEQ_BLOCK, dim), lambda i, j: (j, i, 0)),
        out_shape=jax.ShapeDtypeStruct(tensor_in.shape, tensor_in.dtype),
    )(tensor_in, pos_table)
